# baseline (speedup 1.0000x reference)
import functools

import jax
import jax.numpy as jnp
from jax import lax
from jax.experimental import pallas as pl
from jax.experimental.pallas import tpu as pltpu
from jax.experimental.pallas import tpu_sc as plsc

_B = 16384
_NS = 20
_D = 128
_C = 64
_NBUF = 10
_L = 16


@functools.cache
def _build_gather_kernel():
  info = plsc.get_sparse_core_info()
  nc, nsub = info.num_cores, info.num_subcores
  nw = nc * nsub

  n_small = _B // nw
  n_noise = (_B * _NS) // nw
  nch_noise = n_noise // _C
  assert nch_noise % _NBUF == 0 and n_small % _C == 0

  mesh = plsc.VectorSubcoreMesh(core_axis_name="c", subcore_axis_name="s")

  @functools.partial(
      pl.kernel,
      out_type=[
          jax.ShapeDtypeStruct((_B, _D), jnp.float32),
          jax.ShapeDtypeStruct((_B, _D), jnp.float32),
          jax.ShapeDtypeStruct((_B * _NS, _D), jnp.float32),
      ],
      mesh=mesh,
      scratch_types=(
          [pltpu.VMEM((nch_noise, _C), jnp.int32),
           pltpu.VMEM((nch_noise, _C), jnp.int32),
           pltpu.VMEM((n_small // _C, _C), jnp.int32)]
          + [pltpu.VMEM((_C, _D), jnp.float32) for _ in range(_NBUF)]
          + [pltpu.SemaphoreType.DMA for _ in range(2 * _NBUF + 1)]
      ),
  )
  def gather_kernel(iw_hbm, ow_hbm, nw_hbm, ie_hbm, oe_hbm, o1, o2, o3, *scr):
    idxrows = scr[0]
    outrows = scr[1]
    sidx = scr[2]
    rows = scr[3:3 + _NBUF]
    gsem = scr[3 + _NBUF:3 + 2 * _NBUF]
    wsem = scr[3 + 2 * _NBUF:3 + 3 * _NBUF]
    nsem = scr[3 + 3 * _NBUF]
    wid = lax.axis_index("s") * nc + lax.axis_index("c")
    lanes = lax.iota(jnp.int32, _L)

    pltpu.async_copy(nw_hbm.at[pl.ds(wid * nch_noise, nch_noise)], idxrows,
                     nsem)

    def small_phase(idx2_hbm, tab_hbm, out_hbm):
      nch = n_small // _C
      base = wid * n_small
      pltpu.sync_copy(idx2_hbm.at[pl.ds(wid * nch, nch)], sidx)
      for b in range(nch):
        pltpu.async_copy(tab_hbm.at[sidx.at[b]], rows[b], gsem[b])
      for b in range(nch):
        out_slice = out_hbm.at[pl.ds(base + b * _C, _C)]
        pltpu.make_async_copy(tab_hbm.at[sidx.at[b]], rows[b], gsem[b]).wait()
        pltpu.async_copy(rows[b], out_slice, wsem[b])
      for b in range(nch):
        pltpu.make_async_copy(rows[b], out_hbm.at[pl.ds(base, _C)], wsem[b]).wait()

    small_phase(iw_hbm, ie_hbm, o1)
    small_phase(ow_hbm, oe_hbm, o2)

    def orow(r, carry):
      nvec, bvec = carry
      for j in range(_C // _L):
        outrows[r, pl.ds(j * _L, _L)] = nvec * _B + bvec
        nxt = nvec + _L
        wrap = nxt >= _NS
        nvec = jnp.where(wrap, nxt - _NS, nxt)
        bvec = jnp.where(wrap, bvec + 1, bvec)
      return (nvec, bvec)

    lax.fori_loop(0, nch_noise, orow,
                  (lanes, jnp.full((_L,), wid * (_B // nw), jnp.int32)))

    pltpu.make_async_copy(nw_hbm.at[pl.ds(wid * nch_noise, nch_noise)],
                          idxrows, nsem).wait()
    for b in range(_NBUF):
      pltpu.async_copy(oe_hbm.at[idxrows.at[b]], rows[b], gsem[b])

    def step(s, carry):
      for b in range(_NBUF):
        g = s * _NBUF + b
        pltpu.make_async_copy(oe_hbm.at[idxrows.at[g]], rows[b], gsem[b]).wait()
        pltpu.async_copy(rows[b], o3.at[outrows.at[g]], wsem[b])

        @pl.when(s < (nch_noise // _NBUF) - 1)
        def _():
          pltpu.make_async_copy(rows[b], o3.at[outrows.at[g]], wsem[b]).wait()
          pltpu.async_copy(oe_hbm.at[idxrows.at[g + _NBUF]], rows[b], gsem[b])

      return carry

    lax.fori_loop(0, nch_noise // _NBUF, step, 0)
    for b in range(_NBUF):
      pltpu.make_async_copy(rows[b], o3.at[outrows.at[nch_noise - _NBUF + b]],
                            wsem[b]).wait()

  return gather_kernel


def kernel(input_words, output_words, noise_words, in_embeddings, out_embeddings):
  gather = _build_gather_kernel()
  o1, o2, o3 = gather(
      input_words.astype(jnp.int32).reshape(-1, _C),
      output_words.astype(jnp.int32).reshape(-1, _C),
      noise_words.astype(jnp.int32).reshape(-1, _C),
      in_embeddings,
      out_embeddings,
  )
  return (o1, o2, jnp.swapaxes(o3.reshape(_NS, _B, _D), 0, 1))


# speedup vs baseline: 1.0223x; 1.0223x over previous
import functools

import jax
import jax.numpy as jnp
from jax import lax
from jax.experimental import pallas as pl
from jax.experimental.pallas import tpu as pltpu
from jax.experimental.pallas import tpu_sc as plsc

_B = 16384
_NS = 20
_D = 128
_C = 128
_NBUF = 5
_L = 16


@functools.cache
def _build_gather_kernel():
  info = plsc.get_sparse_core_info()
  nc, nsub = info.num_cores, info.num_subcores
  nw = nc * nsub

  n_small = _B // nw
  n_noise = (_B * _NS) // nw
  nch_small = n_small // _C
  nch_noise = n_noise // _C
  nch = 2 * nch_small + nch_noise
  nfull = nch // _NBUF
  nrem = nch % _NBUF
  g_noise = 2 * nch_small
  assert nrem < _NBUF and nfull * _NBUF + nrem == nch
  assert (nfull - 1) * _NBUF + _NBUF - 1 >= g_noise

  mesh = plsc.VectorSubcoreMesh(core_axis_name="c", subcore_axis_name="s")

  @functools.partial(
      pl.kernel,
      out_type=[
          jax.ShapeDtypeStruct((_B, _D), jnp.float32),
          jax.ShapeDtypeStruct((_B, _D), jnp.float32),
          jax.ShapeDtypeStruct((_B * _NS, _D), jnp.float32),
      ],
      mesh=mesh,
      scratch_types=(
          [pltpu.VMEM((nch_noise, _C), jnp.int32),
           pltpu.VMEM((nch_noise, _C), jnp.int32),
           pltpu.VMEM((2 * nch_small, _C), jnp.int32)]
          + [pltpu.VMEM((_C, _D), jnp.float32) for _ in range(_NBUF)]
          + [pltpu.SemaphoreType.DMA for _ in range(2 * _NBUF + 1)]
      ),
  )
  def gather_kernel(iw_hbm, ow_hbm, nw_hbm, ie_hbm, oe_hbm, o1, o2, o3, *scr):
    idxrows = scr[0]
    outrows = scr[1]
    sidx = scr[2]
    rows = scr[3:3 + _NBUF]
    gsem = scr[3 + _NBUF:3 + 2 * _NBUF]
    wsem = scr[3 + 2 * _NBUF:3 + 3 * _NBUF]
    nsem = scr[3 + 3 * _NBUF]
    wid = lax.axis_index("s") * nc + lax.axis_index("c")
    lanes = lax.iota(jnp.int32, _L)

    pltpu.async_copy(nw_hbm.at[pl.ds(wid * nch_noise, nch_noise)], idxrows,
                     nsem)
    pltpu.sync_copy(iw_hbm.at[pl.ds(wid * nch_small, nch_small)],
                    sidx.at[pl.ds(0, nch_small)])
    pltpu.sync_copy(ow_hbm.at[pl.ds(wid * nch_small, nch_small)],
                    sidx.at[pl.ds(nch_small, nch_small)])

    def orow(r, carry):
      nvec, bvec = carry
      for j in range(_C // _L):
        outrows[r, pl.ds(j * _L, _L)] = nvec * _B + bvec
        nxt = nvec + _L
        wrap = nxt >= _NS
        nvec = jnp.where(wrap, nxt - _NS, nxt)
        bvec = jnp.where(wrap, bvec + 1, bvec)
      return (nvec, bvec)

    lax.fori_loop(0, nch_noise, orow,
                  (lanes, jnp.full((_L,), wid * (_B // nw), jnp.int32)))

    def issue_gather_dyn(gg, b):
      @pl.when(gg < g_noise)
      def _():
        pltpu.async_copy(oe_hbm.at[sidx.at[gg]], rows[b], gsem[b])

      @pl.when(gg >= g_noise)
      def _():
        pltpu.async_copy(oe_hbm.at[idxrows.at[gg - g_noise]], rows[b], gsem[b])

    def wait_gather(b):
      pltpu.make_async_copy(oe_hbm.at[idxrows.at[0]], rows[b], gsem[b]).wait()

    def start_write(gg, b):
      @pl.when(gg < nch_small)
      def _():
        pltpu.async_copy(rows[b],
                         o1.at[pl.ds(wid * n_small + gg * _C, _C)], wsem[b])

      @pl.when(jnp.logical_and(gg >= nch_small, gg < g_noise))
      def _():
        pltpu.async_copy(
            rows[b],
            o2.at[pl.ds(wid * n_small + (gg - nch_small) * _C, _C)], wsem[b])

      @pl.when(gg >= g_noise)
      def _():
        pltpu.async_copy(rows[b], o3.at[outrows.at[gg - g_noise]], wsem[b])

    def wait_write(gg, b):
      @pl.when(gg < g_noise)
      def _():
        pltpu.make_async_copy(rows[b], o1.at[pl.ds(wid * n_small, _C)],
                              wsem[b]).wait()

      @pl.when(gg >= g_noise)
      def _():
        pltpu.make_async_copy(rows[b], o3.at[outrows.at[0]], wsem[b]).wait()

    for g in range(_NBUF):
      tab = ie_hbm if g < nch_small else oe_hbm
      pltpu.async_copy(tab.at[sidx.at[g]], rows[g], gsem[g])
    pltpu.make_async_copy(nw_hbm.at[pl.ds(wid * nch_noise, nch_noise)],
                          idxrows, nsem).wait()

    def step(s, carry):
      for b in range(_NBUF):
        g = s * _NBUF + b
        wait_gather(b)
        start_write(g, b)

        @pl.when(g + _NBUF < nch)
        def _():
          wait_write(g, b)
          issue_gather_dyn(g + _NBUF, b)

      return carry

    lax.fori_loop(0, nfull, step, 0)
    for b in range(nrem):
      g = nfull * _NBUF + b
      wait_gather(b)
      pltpu.async_copy(rows[b], o3.at[outrows.at[g - g_noise]], wsem[b])
    for b in range(_NBUF):
      g = nfull * _NBUF + b if b < nrem else (nfull - 1) * _NBUF + b
      pltpu.make_async_copy(rows[b], o3.at[outrows.at[g - g_noise]],
                            wsem[b]).wait()

  return gather_kernel


def kernel(input_words, output_words, noise_words, in_embeddings, out_embeddings):
  gather = _build_gather_kernel()
  o1, o2, o3 = gather(
      input_words.astype(jnp.int32).reshape(-1, _C),
      output_words.astype(jnp.int32).reshape(-1, _C),
      noise_words.astype(jnp.int32).reshape(-1, _C),
      in_embeddings,
      out_embeddings,
  )
  return (o1, o2, jnp.swapaxes(o3.reshape(_NS, _B, _D), 0, 1))
